# baseline (speedup 1.0000x reference)
import functools

import jax
import jax.numpy as jnp
from jax import lax
from jax.experimental import pallas as pl
from jax.experimental.pallas import tpu as pltpu
from jax.experimental.pallas import tpu_sc as plsc

BATCH = 4096
SEQ_LEN = 50
EMBED_DIM = 64
ROWS = BATCH * SEQ_LEN
NC, NS = 2, 16
NW = NC * NS
ROWS_PER_W = ROWS // NW
CHUNK = 128
NCHUNK = ROWS_PER_W // CHUNK


@functools.cache
def _make_gather():
    mesh = plsc.VectorSubcoreMesh(core_axis_name="c", subcore_axis_name="s",
                                  num_cores=NC, num_subcores=NS)

    @functools.partial(
        pl.kernel,
        mesh=mesh,
        out_type=jax.ShapeDtypeStruct((ROWS, EMBED_DIM), jnp.float32),
        scratch_types=[
            pltpu.VMEM((NCHUNK, CHUNK), jnp.int32),
            pltpu.VMEM((CHUNK, EMBED_DIM), jnp.float32),
            pltpu.SemaphoreType.DMA,
        ],
        compiler_params=pltpu.CompilerParams(use_tc_tiling_on_sc=False),
    )
    def gather_k(ids_hbm, table_hbm, out_hbm, idx_v, rows_v, sem):
        wid = lax.axis_index("s") * NC + lax.axis_index("c")
        pltpu.sync_copy(ids_hbm.at[wid], idx_v)
        base = wid * ROWS_PER_W

        def body(j, _):
            pltpu.async_copy(table_hbm.at[idx_v.at[j]], rows_v, sem).wait()
            pltpu.sync_copy(rows_v, out_hbm.at[pl.ds(base + j * CHUNK, CHUNK)])
            return 0

        lax.fori_loop(0, NCHUNK, body, 0)

    return gather_k


_BLK = 2048


def _mm_body(ue_ref, soc_ref, w1_ref, w2_ref, b_ref, out_ref):
    acc = jnp.dot(ue_ref[...], w1_ref[...], preferred_element_type=jnp.float32)
    acc += jnp.dot(soc_ref[...], w2_ref[...], preferred_element_type=jnp.float32)
    out_ref[...] = jnp.maximum(acc + b_ref[...], 0.0)


def _fused_linear(ue, soc, w1t, w2t, b2d):
    return pl.pallas_call(
        _mm_body,
        grid=(ROWS // _BLK,),
        in_specs=[
            pl.BlockSpec((_BLK, EMBED_DIM), lambda i: (i, 0)),
            pl.BlockSpec((_BLK, EMBED_DIM), lambda i: (i, 0)),
            pl.BlockSpec((EMBED_DIM, EMBED_DIM), lambda i: (0, 0)),
            pl.BlockSpec((EMBED_DIM, EMBED_DIM), lambda i: (0, 0)),
            pl.BlockSpec((1, EMBED_DIM), lambda i: (0, 0)),
        ],
        out_specs=pl.BlockSpec((_BLK, EMBED_DIM), lambda i: (i, 0)),
        out_shape=jax.ShapeDtypeStruct((ROWS, EMBED_DIM), jnp.float32),
    )(ue, soc, w1t, w2t, b2d)


def kernel(user_embeds, user_ids, emb_table, W, b):
    ids = user_ids.astype(jnp.int32).reshape(NW, NCHUNK, CHUNK)
    social = _make_gather()(ids, emb_table)
    ue = user_embeds.reshape(ROWS, EMBED_DIM)
    wt = W.T
    out = _fused_linear(ue, social, wt[:EMBED_DIM], wt[EMBED_DIM:],
                        b.reshape(1, EMBED_DIM))
    return out.reshape(BATCH, SEQ_LEN, EMBED_DIM)
